# Initial kernel scaffold; baseline (speedup 1.0000x reference)
import jax
import jax.numpy as jnp
from jax.experimental import pallas as pl


def kernel(state, edge_index, action, Wc, bc, W1, b1, W2, b2, W3, b3):
    raise NotImplementedError("write your pallas kernel here")


# speedup vs baseline: 18.2593x; 18.2593x over previous
import functools

import jax
import jax.numpy as jnp
from jax import lax
from jax.experimental import pallas as pl
from jax.experimental.pallas import tpu as pltpu
from jax.experimental.pallas import tpu_sc as plsc

_ACT = 14
_CH = 128
_HID = 64
_NC = 2
_NS = 16
_NW = _NC * _NS
_CK = 128
_DEGW = 16


def _sc_mesh():
    return plsc.VectorSubcoreMesh(
        core_axis_name="c", subcore_axis_name="s",
        num_cores=_NC, num_subcores=_NS)


def _make_deg_kernel(N, E):
    ept = E // _NW
    niter = ept // _CK
    rpt = N // _NS

    @functools.partial(
        pl.kernel,
        out_type=jax.ShapeDtypeStruct((_NC, N, _DEGW), jnp.float32),
        mesh=_sc_mesh(),
        scratch_types=[
            pltpu.VMEM((_CK,), jnp.int32),
            pltpu.VMEM((_CK, _DEGW), jnp.float32),
            pltpu.VMEM_SHARED((N, _DEGW), jnp.float32),
        ],
        compiler_params=pltpu.CompilerParams(use_tc_tiling_on_sc=False),
    )
    def deg_kernel(dst_hbm, ones_hbm, zeros_hbm, out_hbm, idx_v, ones_v, deg_sh):
        c = lax.axis_index("c")
        s = lax.axis_index("s")
        wid = c * _NS + s
        pltpu.sync_copy(zeros_hbm.at[pl.ds(s * rpt, rpt)],
                        deg_sh.at[pl.ds(s * rpt, rpt)])
        pltpu.sync_copy(ones_hbm, ones_v)
        plsc.subcore_barrier()
        base = wid * ept

        @pl.loop(0, niter)
        def _(it):
            off = base + it * _CK
            pltpu.sync_copy(dst_hbm.at[pl.ds(off, _CK)], idx_v)
            pltpu.sync_copy(ones_v, deg_sh.at[idx_v], add=True)

        plsc.subcore_barrier()
        pltpu.sync_copy(deg_sh.at[pl.ds(s * rpt, rpt)],
                        out_hbm.at[c, pl.ds(s * rpt, rpt)])

    return deg_kernel


_HCH = _CH // 2


def _make_scatter_kernel(N, E):
    ept = E // _NW
    niter = ept // _CK
    rpt = N // _NS

    @functools.partial(
        pl.kernel,
        out_type=[jax.ShapeDtypeStruct((_NC, N, _HCH), jnp.float32),
                  jax.ShapeDtypeStruct((_NC, N, _HCH), jnp.float32)],
        mesh=_sc_mesh(),
        scratch_types=[
            pltpu.VMEM((_CK,), jnp.int32),
            pltpu.VMEM((_CK,), jnp.int32),
            pltpu.VMEM((_CK, _HCH), jnp.float32),
            pltpu.VMEM_SHARED((N, _HCH), jnp.float32),
            pltpu.SemaphoreType.DMA,
        ],
        compiler_params=pltpu.CompilerParams(use_tc_tiling_on_sc=False),
    )
    def scatter_kernel(ya_hbm, yb_hbm, src_hbm, dst_hbm, zeros_hbm,
                       outa_hbm, outb_hbm, sidx_v, didx_v, rows_v, s_sh, sem):
        c = lax.axis_index("c")
        s = lax.axis_index("s")
        wid = c * _NS + s
        base = wid * ept
        for y_hbm, out_hbm in ((ya_hbm, outa_hbm), (yb_hbm, outb_hbm)):
            pltpu.sync_copy(zeros_hbm.at[pl.ds(s * rpt, rpt)],
                            s_sh.at[pl.ds(s * rpt, rpt)])
            plsc.subcore_barrier()

            @pl.loop(0, niter)
            def _(it):
                off = base + it * _CK
                pltpu.sync_copy(src_hbm.at[pl.ds(off, _CK)], sidx_v)
                pltpu.async_copy(y_hbm.at[sidx_v], rows_v, sem).wait()
                pltpu.sync_copy(dst_hbm.at[pl.ds(off, _CK)], didx_v)
                pltpu.sync_copy(rows_v, s_sh.at[didx_v], add=True)

            plsc.subcore_barrier()
            pltpu.sync_copy(s_sh.at[pl.ds(s * rpt, rpt)],
                            out_hbm.at[c, pl.ds(s * rpt, rpt)])
            plsc.subcore_barrier()

    return scatter_kernel


def _tc1_body(state_ref, wc_ref, degp_ref, y_ref, ya_ref, yb_ref, dis_ref):
    deg = jnp.sum(degp_ref[...], axis=1, keepdims=True) * (1.0 / _DEGW) + 1.0
    dis = lax.rsqrt(deg)
    xw = jnp.dot(state_ref[...], wc_ref[...], preferred_element_type=jnp.float32)
    y = xw * dis
    y_ref[...] = y
    ya_ref[...] = y[:, :_HCH]
    yb_ref[...] = y[:, _HCH:]
    dis_ref[...] = dis


def _lrelu(v):
    return jnp.where(v > 0, v, 0.01 * v)


def _tc2_body(sa0_ref, sb0_ref, sa1_ref, sb1_ref, y_ref, st_ref, dis_ref,
              act_ref, bc_ref,
              w1a_ref, w1b_ref, w1c_ref, b1_ref, w2_ref, b2_ref, w3_ref, b3_ref,
              out_ref, p_s, q_s):
    w1a = w1a_ref[...]
    w1b = w1b_ref[...]
    bc = bc_ref[...]
    for i in range(_ACT):
        pre_a = sa0_ref[:, i, :] + sa1_ref[:, i, :] + y_ref[:, i, :_HCH]
        pre_b = sb0_ref[:, i, :] + sb1_ref[:, i, :] + y_ref[:, i, _HCH:]
        pre = jnp.concatenate([pre_a, pre_b], axis=1)
        xi = jnp.maximum(
            pre * dis_ref[:, i, :] + bc, 0.0) + st_ref[:, i, :]
        p_s[i, :, :] = jnp.dot(xi, w1a, preferred_element_type=jnp.float32)
        q_s[i, :, :] = jnp.dot(xi, w1b, preferred_element_type=jnp.float32)
    w1c = w1c_ref[...]
    b1 = b1_ref[...]
    w2 = w2_ref[...]
    b2 = b2_ref[...]
    nb = out_ref.shape[0]
    acc = jnp.zeros((nb, _HID), jnp.float32)
    for i in range(_ACT):
        pre = p_s[i, :, :] + q_s[(i + 1) % _ACT, :, :] \
            + act_ref[:, i:i + 1] * w1c + b1
        h = jnp.dot(_lrelu(pre), w2, preferred_element_type=jnp.float32) + b2
        acc = acc + _lrelu(h)
        pre = p_s[i, :, :] + q_s[(i + 2) % _ACT, :, :] \
            + act_ref[:, _ACT + i:_ACT + i + 1] * w1c + b1
        h = jnp.dot(_lrelu(pre), w2, preferred_element_type=jnp.float32) + b2
        acc = acc + _lrelu(h)
    out_ref[...] = jnp.dot(acc, w3_ref[...],
                           preferred_element_type=jnp.float32) + b3_ref[...]


def kernel(state, edge_index, action, Wc, bc, W1, b1, W2, b2, W3, b3):
    N, ch = state.shape
    E = edge_index.shape[1]
    B = action.shape[0]
    src = edge_index[0]
    dst = edge_index[1]

    deg_parts = _make_deg_kernel(N, E)(
        dst,
        jnp.ones((_CK, _DEGW), jnp.float32),
        jnp.zeros((N, _DEGW), jnp.float32),
    )
    degp = jnp.moveaxis(deg_parts, 0, 1).reshape(N, _NC * _DEGW)

    rb = 1792
    grid1 = N // rb
    y, ya, yb, dis = pl.pallas_call(
        _tc1_body,
        grid=(grid1,),
        in_specs=[
            pl.BlockSpec((rb, ch), lambda i: (i, 0)),
            pl.BlockSpec((ch, ch), lambda i: (0, 0)),
            pl.BlockSpec((rb, _NC * _DEGW), lambda i: (i, 0)),
        ],
        out_specs=[
            pl.BlockSpec((rb, ch), lambda i: (i, 0)),
            pl.BlockSpec((rb, _HCH), lambda i: (i, 0)),
            pl.BlockSpec((rb, _HCH), lambda i: (i, 0)),
            pl.BlockSpec((rb, 1), lambda i: (i, 0)),
        ],
        out_shape=[
            jax.ShapeDtypeStruct((N, ch), jnp.float32),
            jax.ShapeDtypeStruct((N, _HCH), jnp.float32),
            jax.ShapeDtypeStruct((N, _HCH), jnp.float32),
            jax.ShapeDtypeStruct((N, 1), jnp.float32),
        ],
    )(state, Wc, degp)

    s_a, s_b = _make_scatter_kernel(N, E)(
        ya, yb, src, dst, jnp.zeros((N, _HCH), jnp.float32))

    bb = 128
    grid2 = B // bb
    w1a = W1[:ch]
    w1b = W1[ch:2 * ch]
    w1c = W1[2 * ch:]
    out = pl.pallas_call(
        _tc2_body,
        grid=(grid2,),
        in_specs=[
            pl.BlockSpec((bb, _ACT, _HCH), lambda i: (i, 0, 0)),
            pl.BlockSpec((bb, _ACT, _HCH), lambda i: (i, 0, 0)),
            pl.BlockSpec((bb, _ACT, _HCH), lambda i: (i, 0, 0)),
            pl.BlockSpec((bb, _ACT, _HCH), lambda i: (i, 0, 0)),
            pl.BlockSpec((bb, _ACT, ch), lambda i: (i, 0, 0)),
            pl.BlockSpec((bb, _ACT, ch), lambda i: (i, 0, 0)),
            pl.BlockSpec((bb, _ACT, 1), lambda i: (i, 0, 0)),
            pl.BlockSpec((bb, 2 * _ACT), lambda i: (i, 0)),
            pl.BlockSpec((1, ch), lambda i: (0, 0)),
            pl.BlockSpec((ch, _HID), lambda i: (0, 0)),
            pl.BlockSpec((ch, _HID), lambda i: (0, 0)),
            pl.BlockSpec((1, _HID), lambda i: (0, 0)),
            pl.BlockSpec((1, _HID), lambda i: (0, 0)),
            pl.BlockSpec((_HID, _HID), lambda i: (0, 0)),
            pl.BlockSpec((1, _HID), lambda i: (0, 0)),
            pl.BlockSpec((_HID, 1), lambda i: (0, 0)),
            pl.BlockSpec((1, 1), lambda i: (0, 0)),
        ],
        out_specs=pl.BlockSpec((bb, 1), lambda i: (i, 0)),
        out_shape=jax.ShapeDtypeStruct((B, 1), jnp.float32),
        scratch_shapes=[
            pltpu.VMEM((_ACT, bb, _HID), jnp.float32),
            pltpu.VMEM((_ACT, bb, _HID), jnp.float32),
        ],
    )(s_a[0].reshape(B, _ACT, _HCH),
      s_b[0].reshape(B, _ACT, _HCH),
      s_a[1].reshape(B, _ACT, _HCH),
      s_b[1].reshape(B, _ACT, _HCH),
      y.reshape(B, _ACT, ch),
      state.reshape(B, _ACT, ch),
      dis.reshape(B, _ACT, 1),
      action,
      bc.reshape(1, ch),
      w1a, w1b, w1c,
      b1.reshape(1, _HID),
      W2,
      b2.reshape(1, _HID),
      W3,
      b3.reshape(1, 1))
    return out[:, 0]


# speedup vs baseline: 40.0595x; 2.1939x over previous
import functools

import jax
import jax.numpy as jnp
from jax import lax
from jax.experimental import pallas as pl
from jax.experimental.pallas import tpu as pltpu
from jax.experimental.pallas import tpu_sc as plsc

_ACT = 14
_CH = 128
_HID = 64
_NC = 2
_NS = 16
_NW = _NC * _NS
_CK = 128
_DEGW = 16


def _sc_mesh():
    return plsc.VectorSubcoreMesh(
        core_axis_name="c", subcore_axis_name="s",
        num_cores=_NC, num_subcores=_NS)


def _make_deg_kernel(N, E):
    ept = E // _NW
    niter = ept // _CK
    rpt = N // _NS

    @functools.partial(
        pl.kernel,
        out_type=jax.ShapeDtypeStruct((_NC, N, _DEGW), jnp.float32),
        mesh=_sc_mesh(),
        scratch_types=[
            pltpu.VMEM((3, _CK), jnp.int32),
            pltpu.VMEM((_CK, _DEGW), jnp.float32),
            pltpu.VMEM_SHARED((N, _DEGW), jnp.float32),
            pltpu.SemaphoreType.DMA,
            pltpu.SemaphoreType.DMA,
        ],
        compiler_params=pltpu.CompilerParams(use_tc_tiling_on_sc=False),
    )
    def deg_kernel(dst_hbm, ones_hbm, zeros_hbm, out_hbm, idx_v, ones_v,
                   deg_sh, isem, asem):
        c = lax.axis_index("c")
        s = lax.axis_index("s")
        wid = c * _NS + s
        pltpu.sync_copy(zeros_hbm.at[pl.ds(s * rpt, rpt)],
                        deg_sh.at[pl.ds(s * rpt, rpt)])
        pltpu.sync_copy(ones_hbm, ones_v)
        plsc.subcore_barrier()
        base = wid * ept

        def istart(g, b):
            pltpu.async_copy(dst_hbm.at[pl.ds(base + g * _CK, _CK)],
                             idx_v.at[b], isem)

        def iwait(b):
            pltpu.make_async_copy(dst_hbm.at[pl.ds(0, _CK)],
                                  idx_v.at[b], isem).wait()

        def astart(b):
            pltpu.async_copy(ones_v, deg_sh.at[idx_v.at[b]], asem, add=True)

        def await_(b):
            pltpu.make_async_copy(ones_v, deg_sh.at[idx_v.at[b]], asem).wait()

        istart(0, 0)
        iwait(0)
        istart(1, 1)
        iwait(1)

        @pl.loop(0, niter)
        def _(g):
            b = lax.rem(g, 3)
            astart(b)

            @pl.when(g >= 1)
            def _():
                await_(lax.rem(g + 2, 3))

            @pl.when(g + 2 < niter)
            def _():
                b2 = lax.rem(g + 2, 3)
                istart(g + 2, b2)
                iwait(b2)

        await_(lax.rem(niter - 1, 3))
        plsc.subcore_barrier()
        pltpu.sync_copy(deg_sh.at[pl.ds(s * rpt, rpt)],
                        out_hbm.at[c, pl.ds(s * rpt, rpt)])

    return deg_kernel


_HCH = _CH // 2


def _make_scatter_kernel(N, E):
    ept = E // _NS
    niter = ept // _CK
    rpt = N // _NS

    @functools.partial(
        pl.kernel,
        out_type=jax.ShapeDtypeStruct((N, _CH), jnp.float32),
        mesh=_sc_mesh(),
        scratch_types=[
            pltpu.VMEM((3, _CK), jnp.int32),
            pltpu.VMEM((3, _CK), jnp.int32),
            pltpu.VMEM((3, _CK, _HCH), jnp.float32),
            pltpu.VMEM_SHARED((N, _HCH), jnp.float32),
            pltpu.SemaphoreType.DMA,
            pltpu.SemaphoreType.DMA,
            pltpu.SemaphoreType.DMA,
        ],
        compiler_params=pltpu.CompilerParams(use_tc_tiling_on_sc=False),
    )
    def scatter_kernel(ya_hbm, yb_hbm, src_hbm, dst_hbm, zeros_hbm, out_hbm,
                       sidx_v, didx_v, rows_v, s_sh, isem, gsem, asem):
        c = lax.axis_index("c")
        s = lax.axis_index("s")
        pltpu.sync_copy(zeros_hbm.at[pl.ds(s * rpt, rpt)],
                        s_sh.at[pl.ds(s * rpt, rpt)])
        plsc.subcore_barrier()
        base = s * ept

        def run(y_hbm):
            def istart(g, b):
                off = base + g * _CK
                pltpu.async_copy(src_hbm.at[pl.ds(off, _CK)],
                                 sidx_v.at[b], isem)
                pltpu.async_copy(dst_hbm.at[pl.ds(off, _CK)],
                                 didx_v.at[b], isem)

            def iwait(b):
                pltpu.make_async_copy(src_hbm.at[pl.ds(0, _CK)],
                                      sidx_v.at[b], isem).wait()
                pltpu.make_async_copy(dst_hbm.at[pl.ds(0, _CK)],
                                      didx_v.at[b], isem).wait()

            def gstart(b):
                pltpu.async_copy(y_hbm.at[sidx_v.at[b]], rows_v.at[b], gsem)

            def gwait(b):
                pltpu.make_async_copy(y_hbm.at[sidx_v.at[b]],
                                      rows_v.at[b], gsem).wait()

            def astart(b):
                pltpu.async_copy(rows_v.at[b], s_sh.at[didx_v.at[b]],
                                 asem, add=True)

            def await_(b):
                pltpu.make_async_copy(rows_v.at[b], s_sh.at[didx_v.at[b]],
                                      asem).wait()

            istart(0, 0)
            iwait(0)
            gstart(0)
            istart(1, 1)
            iwait(1)
            gstart(1)

            @pl.loop(0, niter)
            def _(g):
                b = lax.rem(g, 3)
                gwait(b)
                astart(b)

                @pl.when(g >= 1)
                def _():
                    await_(lax.rem(g + 2, 3))

                @pl.when(g + 2 < niter)
                def _():
                    b2 = lax.rem(g + 2, 3)
                    istart(g + 2, b2)
                    iwait(b2)
                    gstart(b2)

            await_(lax.rem(niter - 1, 3))

        run_sel = [ya_hbm, yb_hbm]
        for ci in range(_NC):
            @pl.when(c == ci)
            def _(ci=ci):
                run(run_sel[ci])
        plsc.subcore_barrier()
        for ci in range(_NC):
            @pl.when(c == ci)
            def _(ci=ci):
                pltpu.sync_copy(
                    s_sh.at[pl.ds(s * rpt, rpt)],
                    out_hbm.at[pl.ds(s * rpt, rpt), pl.ds(ci * _HCH, _HCH)])

    return scatter_kernel


def _tc1_body(state_ref, wc_ref, degp_ref, y_ref, ya_ref, yb_ref, dis_ref):
    deg = jnp.sum(degp_ref[...], axis=1, keepdims=True) * (1.0 / _DEGW) + 1.0
    dis = lax.rsqrt(deg)
    xw = jnp.dot(state_ref[...], wc_ref[...], preferred_element_type=jnp.float32)
    y = xw * dis
    y_ref[...] = y
    ya_ref[...] = y[:, :_HCH]
    yb_ref[...] = y[:, _HCH:]
    dis_ref[...] = dis


def _lrelu(v):
    return jnp.where(v > 0, v, 0.01 * v)


def _tc2_body(s_ref, y_ref, st_ref, dis_ref, act_ref, bc_ref,
              w1a_ref, w1b_ref, w1c_ref, b1_ref, w2_ref, b2_ref, w3_ref, b3_ref,
              out_ref, p_s, q_s):
    w1a = w1a_ref[...]
    w1b = w1b_ref[...]
    bc = bc_ref[...]
    for i in range(_ACT):
        pre = s_ref[:, i, :] + y_ref[:, i, :]
        xi = jnp.maximum(
            pre * dis_ref[:, i, :] + bc, 0.0) + st_ref[:, i, :]
        p_s[i, :, :] = jnp.dot(xi, w1a, preferred_element_type=jnp.float32)
        q_s[i, :, :] = jnp.dot(xi, w1b, preferred_element_type=jnp.float32)
    w1c = w1c_ref[...]
    b1 = b1_ref[...]
    w2 = w2_ref[...]
    b2 = b2_ref[...]
    nb = out_ref.shape[0]
    acc = jnp.zeros((nb, _HID), jnp.float32)
    for i in range(_ACT):
        pre = p_s[i, :, :] + q_s[(i + 1) % _ACT, :, :] \
            + act_ref[:, i:i + 1] * w1c + b1
        h = jnp.dot(_lrelu(pre), w2, preferred_element_type=jnp.float32) + b2
        acc = acc + _lrelu(h)
        pre = p_s[i, :, :] + q_s[(i + 2) % _ACT, :, :] \
            + act_ref[:, _ACT + i:_ACT + i + 1] * w1c + b1
        h = jnp.dot(_lrelu(pre), w2, preferred_element_type=jnp.float32) + b2
        acc = acc + _lrelu(h)
    out_ref[...] = jnp.dot(acc, w3_ref[...],
                           preferred_element_type=jnp.float32) + b3_ref[...]


def kernel(state, edge_index, action, Wc, bc, W1, b1, W2, b2, W3, b3):
    N, ch = state.shape
    E = edge_index.shape[1]
    B = action.shape[0]
    src = edge_index[0]
    dst = edge_index[1]

    deg_parts = _make_deg_kernel(N, E)(
        dst,
        jnp.ones((_CK, _DEGW), jnp.float32),
        jnp.zeros((N, _DEGW), jnp.float32),
    )
    degp = jnp.moveaxis(deg_parts, 0, 1).reshape(N, _NC * _DEGW)

    rb = 1792
    grid1 = N // rb
    y, ya, yb, dis = pl.pallas_call(
        _tc1_body,
        grid=(grid1,),
        in_specs=[
            pl.BlockSpec((rb, ch), lambda i: (i, 0)),
            pl.BlockSpec((ch, ch), lambda i: (0, 0)),
            pl.BlockSpec((rb, _NC * _DEGW), lambda i: (i, 0)),
        ],
        out_specs=[
            pl.BlockSpec((rb, ch), lambda i: (i, 0)),
            pl.BlockSpec((rb, _HCH), lambda i: (i, 0)),
            pl.BlockSpec((rb, _HCH), lambda i: (i, 0)),
            pl.BlockSpec((rb, 1), lambda i: (i, 0)),
        ],
        out_shape=[
            jax.ShapeDtypeStruct((N, ch), jnp.float32),
            jax.ShapeDtypeStruct((N, _HCH), jnp.float32),
            jax.ShapeDtypeStruct((N, _HCH), jnp.float32),
            jax.ShapeDtypeStruct((N, 1), jnp.float32),
        ],
    )(state, Wc, degp)

    s_full = _make_scatter_kernel(N, E)(
        ya, yb, src, dst, jnp.zeros((N, _HCH), jnp.float32))

    bb = 128
    grid2 = B // bb
    w1a = W1[:ch]
    w1b = W1[ch:2 * ch]
    w1c = W1[2 * ch:]
    out = pl.pallas_call(
        _tc2_body,
        grid=(grid2,),
        in_specs=[
            pl.BlockSpec((bb, _ACT, ch), lambda i: (i, 0, 0)),
            pl.BlockSpec((bb, _ACT, ch), lambda i: (i, 0, 0)),
            pl.BlockSpec((bb, _ACT, ch), lambda i: (i, 0, 0)),
            pl.BlockSpec((bb, _ACT, 1), lambda i: (i, 0, 0)),
            pl.BlockSpec((bb, 2 * _ACT), lambda i: (i, 0)),
            pl.BlockSpec((1, ch), lambda i: (0, 0)),
            pl.BlockSpec((ch, _HID), lambda i: (0, 0)),
            pl.BlockSpec((ch, _HID), lambda i: (0, 0)),
            pl.BlockSpec((1, _HID), lambda i: (0, 0)),
            pl.BlockSpec((1, _HID), lambda i: (0, 0)),
            pl.BlockSpec((_HID, _HID), lambda i: (0, 0)),
            pl.BlockSpec((1, _HID), lambda i: (0, 0)),
            pl.BlockSpec((_HID, 1), lambda i: (0, 0)),
            pl.BlockSpec((1, 1), lambda i: (0, 0)),
        ],
        out_specs=pl.BlockSpec((bb, 1), lambda i: (i, 0)),
        out_shape=jax.ShapeDtypeStruct((B, 1), jnp.float32),
        scratch_shapes=[
            pltpu.VMEM((_ACT, bb, _HID), jnp.float32),
            pltpu.VMEM((_ACT, bb, _HID), jnp.float32),
        ],
    )(s_full.reshape(B, _ACT, ch),
      y.reshape(B, _ACT, ch),
      state.reshape(B, _ACT, ch),
      dis.reshape(B, _ACT, 1),
      action,
      bc.reshape(1, ch),
      w1a, w1b, w1c,
      b1.reshape(1, _HID),
      W2,
      b2.reshape(1, _HID),
      W3,
      b3.reshape(1, 1))
    return out[:, 0]


# speedup vs baseline: 49.2021x; 1.2282x over previous
import functools

import jax
import jax.numpy as jnp
from jax import lax
from jax.experimental import pallas as pl
from jax.experimental.pallas import tpu as pltpu
from jax.experimental.pallas import tpu_sc as plsc

_ACT = 14
_CH = 128
_HID = 64
_NC = 2
_NS = 16
_NW = _NC * _NS
_CK = 128
_DEGW = 16


def _sc_mesh():
    return plsc.VectorSubcoreMesh(
        core_axis_name="c", subcore_axis_name="s",
        num_cores=_NC, num_subcores=_NS)


def _make_deg_kernel(N, E):
    ept = E // _NW
    niter = ept // _CK
    rpt = N // _NS

    @functools.partial(
        pl.kernel,
        out_type=jax.ShapeDtypeStruct((N, _NC * _DEGW), jnp.float32),
        mesh=_sc_mesh(),
        scratch_types=[
            pltpu.VMEM((4, _CK), jnp.int32),
            pltpu.VMEM((_CK, _DEGW), jnp.float32),
            pltpu.VMEM_SHARED((N, _DEGW), jnp.float32),
            pltpu.SemaphoreType.DMA,
            pltpu.SemaphoreType.DMA,
        ],
        compiler_params=pltpu.CompilerParams(use_tc_tiling_on_sc=False),
    )
    def deg_kernel(ei_hbm, ones_hbm, zeros_hbm, out_hbm, idx_v, ones_v,
                   deg_sh, isem, asem):
        c = lax.axis_index("c")
        s = lax.axis_index("s")
        wid = c * _NS + s
        pltpu.sync_copy(zeros_hbm.at[pl.ds(s * rpt, rpt), pl.ds(0, _DEGW)],
                        deg_sh.at[pl.ds(s * rpt, rpt)])
        pltpu.sync_copy(ones_hbm, ones_v)
        plsc.subcore_barrier()
        base = wid * ept

        def istart(g):
            pltpu.async_copy(ei_hbm.at[1, pl.ds(base + g * _CK, _CK)],
                             idx_v.at[lax.rem(g, 4)], isem)

        def iwait(g):
            pltpu.make_async_copy(ei_hbm.at[1, pl.ds(0, _CK)],
                                  idx_v.at[lax.rem(g, 4)], isem).wait()

        def astart(g):
            pltpu.async_copy(ones_v, deg_sh.at[idx_v.at[lax.rem(g, 4)]],
                             asem, add=True)

        def await_(g):
            pltpu.make_async_copy(ones_v, deg_sh.at[idx_v.at[lax.rem(g, 4)]],
                                  asem).wait()

        istart(0)
        istart(1)
        istart(2)
        iwait(0)
        iwait(1)

        @pl.loop(0, niter)
        def _(g):
            astart(g)

            @pl.when(g >= 1)
            def _():
                await_(g - 1)

            @pl.when(g + 2 < niter)
            def _():
                iwait(g + 2)

            @pl.when(g + 3 < niter)
            def _():
                istart(g + 3)

        await_(niter - 1)
        plsc.subcore_barrier()
        pltpu.sync_copy(deg_sh.at[pl.ds(s * rpt, rpt)],
                        out_hbm.at[pl.ds(s * rpt, rpt),
                                   pl.ds(c * _DEGW, _DEGW)])

    return deg_kernel


_HCH = _CH // 2


def _make_scatter_kernel(N, E):
    ept = E // _NS
    niter = ept // _CK
    rpt = N // _NS

    @functools.partial(
        pl.kernel,
        out_type=jax.ShapeDtypeStruct((N, _CH), jnp.float32),
        mesh=_sc_mesh(),
        scratch_types=[
            pltpu.VMEM((4, 2, _CK), jnp.int32),
            pltpu.VMEM((4, _CK), jnp.int32),
            pltpu.VMEM((4, _CK, _HCH), jnp.float32),
            pltpu.VMEM_SHARED((N, _HCH), jnp.float32),
            pltpu.SemaphoreType.DMA,
            pltpu.SemaphoreType.DMA,
            pltpu.SemaphoreType.DMA,
        ],
        compiler_params=pltpu.CompilerParams(use_tc_tiling_on_sc=False),
    )
    def scatter_kernel(y2_hbm, ei_hbm, zeros_hbm, out_hbm,
                       ei_v, tr_v, rows_v, s_sh, isem, gsem, asem):
        c = lax.axis_index("c")
        s = lax.axis_index("s")
        pltpu.sync_copy(zeros_hbm.at[pl.ds(s * rpt, rpt)],
                        s_sh.at[pl.ds(s * rpt, rpt)])
        plsc.subcore_barrier()
        base = s * ept
        c2 = jnp.broadcast_to(c.astype(jnp.int32), (16,))

        def istart(g):
            pltpu.async_copy(ei_hbm.at[:, pl.ds(base + g * _CK, _CK)],
                             ei_v.at[lax.rem(g, 4)], isem)

        def iwait(g):
            pltpu.make_async_copy(ei_hbm.at[:, pl.ds(0, _CK)],
                                  ei_v.at[lax.rem(g, 4)], isem).wait()

        def transform(g):
            b = lax.rem(g, 4)
            for k in range(_CK // 16):
                v = ei_v[b, 0, pl.ds(16 * k, 16)]
                tr_v[b, pl.ds(16 * k, 16)] = v * 2 + c2

        def gstart(g):
            b = lax.rem(g, 4)
            pltpu.async_copy(y2_hbm.at[tr_v.at[b]], rows_v.at[b], gsem)

        def gwait(g):
            b = lax.rem(g, 4)
            pltpu.make_async_copy(y2_hbm.at[tr_v.at[b]],
                                  rows_v.at[b], gsem).wait()

        def astart(g):
            b = lax.rem(g, 4)
            pltpu.async_copy(rows_v.at[b], s_sh.at[ei_v.at[b, 1]],
                             asem, add=True)

        def await_(g):
            b = lax.rem(g, 4)
            pltpu.make_async_copy(rows_v.at[b], s_sh.at[ei_v.at[b, 1]],
                                  asem).wait()

        istart(0)
        istart(1)
        iwait(0)
        transform(0)
        gstart(0)
        istart(2)
        iwait(1)
        transform(1)
        gstart(1)

        @pl.loop(0, niter)
        def _(g):
            gwait(g)
            astart(g)

            @pl.when(g >= 1)
            def _():
                await_(g - 1)

            @pl.when(g + 2 < niter)
            def _():
                iwait(g + 2)
                transform(g + 2)
                gstart(g + 2)

            @pl.when(g + 3 < niter)
            def _():
                istart(g + 3)

        await_(niter - 1)
        plsc.subcore_barrier()
        for ci in range(_NC):
            @pl.when(c == ci)
            def _(ci=ci):
                pltpu.sync_copy(
                    s_sh.at[pl.ds(s * rpt, rpt)],
                    out_hbm.at[pl.ds(s * rpt, rpt), pl.ds(ci * _HCH, _HCH)])

    return scatter_kernel


def _tc1_body(state_ref, wc_ref, degp_ref, y_ref, dis_ref):
    deg = jnp.sum(degp_ref[...], axis=1, keepdims=True) * (1.0 / _DEGW) + 1.0
    dis = lax.rsqrt(deg)
    xw = jnp.dot(state_ref[...], wc_ref[...], preferred_element_type=jnp.float32)
    y_ref[...] = xw * dis
    dis_ref[...] = dis


def _lrelu(v):
    return jnp.where(v > 0, v, 0.01 * v)


def _tc2_body(s_ref, y_ref, st_ref, dis_ref, act_ref, bc_ref,
              w1a_ref, w1b_ref, w1c_ref, b1_ref, w2_ref, b2_ref, w3_ref, b3_ref,
              out_ref, p_s, q_s):
    w1a = w1a_ref[...]
    w1b = w1b_ref[...]
    bc = bc_ref[...]
    for i in range(_ACT):
        pre = s_ref[:, i, :] + y_ref[:, i, :]
        xi = jnp.maximum(
            pre * dis_ref[:, i, :] + bc, 0.0) + st_ref[:, i, :]
        p_s[i, :, :] = jnp.dot(xi, w1a, preferred_element_type=jnp.float32)
        q_s[i, :, :] = jnp.dot(xi, w1b, preferred_element_type=jnp.float32)
    w1c = w1c_ref[...]
    b1 = b1_ref[...]
    w2 = w2_ref[...]
    b2 = b2_ref[...]
    nb = out_ref.shape[0]
    acc = jnp.zeros((nb, _HID), jnp.float32)
    for i in range(_ACT):
        pre = p_s[i, :, :] + q_s[(i + 1) % _ACT, :, :] \
            + act_ref[:, i:i + 1] * w1c + b1
        h = jnp.dot(_lrelu(pre), w2, preferred_element_type=jnp.float32) + b2
        acc = acc + _lrelu(h)
        pre = p_s[i, :, :] + q_s[(i + 2) % _ACT, :, :] \
            + act_ref[:, _ACT + i:_ACT + i + 1] * w1c + b1
        h = jnp.dot(_lrelu(pre), w2, preferred_element_type=jnp.float32) + b2
        acc = acc + _lrelu(h)
    out_ref[...] = jnp.dot(acc, w3_ref[...],
                           preferred_element_type=jnp.float32) + b3_ref[...]


def kernel(state, edge_index, action, Wc, bc, W1, b1, W2, b2, W3, b3):
    N, ch = state.shape
    E = edge_index.shape[1]
    B = action.shape[0]
    zeros = jnp.zeros((N, _HCH), jnp.float32)

    degp = _make_deg_kernel(N, E)(
        edge_index,
        jnp.ones((_CK, _DEGW), jnp.float32),
        zeros,
    )

    rb = 1792
    grid1 = N // rb
    y, dis = pl.pallas_call(
        _tc1_body,
        grid=(grid1,),
        in_specs=[
            pl.BlockSpec((rb, ch), lambda i: (i, 0)),
            pl.BlockSpec((ch, ch), lambda i: (0, 0)),
            pl.BlockSpec((rb, _NC * _DEGW), lambda i: (i, 0)),
        ],
        out_specs=[
            pl.BlockSpec((rb, ch), lambda i: (i, 0)),
            pl.BlockSpec((rb, 1), lambda i: (i, 0)),
        ],
        out_shape=[
            jax.ShapeDtypeStruct((N, ch), jnp.float32),
            jax.ShapeDtypeStruct((N, 1), jnp.float32),
        ],
    )(state, Wc, degp)

    s_full = _make_scatter_kernel(N, E)(
        y.reshape(2 * N, _HCH), edge_index, zeros)

    bb = 128
    grid2 = B // bb
    w1a = W1[:ch]
    w1b = W1[ch:2 * ch]
    w1c = W1[2 * ch:]
    out = pl.pallas_call(
        _tc2_body,
        grid=(grid2,),
        in_specs=[
            pl.BlockSpec((bb, _ACT, ch), lambda i: (i, 0, 0)),
            pl.BlockSpec((bb, _ACT, ch), lambda i: (i, 0, 0)),
            pl.BlockSpec((bb, _ACT, ch), lambda i: (i, 0, 0)),
            pl.BlockSpec((bb, _ACT, 1), lambda i: (i, 0, 0)),
            pl.BlockSpec((bb, 2 * _ACT), lambda i: (i, 0)),
            pl.BlockSpec((1, ch), lambda i: (0, 0)),
            pl.BlockSpec((ch, _HID), lambda i: (0, 0)),
            pl.BlockSpec((ch, _HID), lambda i: (0, 0)),
            pl.BlockSpec((1, _HID), lambda i: (0, 0)),
            pl.BlockSpec((1, _HID), lambda i: (0, 0)),
            pl.BlockSpec((_HID, _HID), lambda i: (0, 0)),
            pl.BlockSpec((1, _HID), lambda i: (0, 0)),
            pl.BlockSpec((_HID, 1), lambda i: (0, 0)),
            pl.BlockSpec((1, 1), lambda i: (0, 0)),
        ],
        out_specs=pl.BlockSpec((bb, 1), lambda i: (i, 0)),
        out_shape=jax.ShapeDtypeStruct((B, 1), jnp.float32),
        scratch_shapes=[
            pltpu.VMEM((_ACT, bb, _HID), jnp.float32),
            pltpu.VMEM((_ACT, bb, _HID), jnp.float32),
        ],
    )(s_full.reshape(B, _ACT, ch),
      y.reshape(B, _ACT, ch),
      state.reshape(B, _ACT, ch),
      dis.reshape(B, _ACT, 1),
      action,
      bc.reshape(1, ch),
      w1a, w1b, w1c,
      b1.reshape(1, _HID),
      W2,
      b2.reshape(1, _HID),
      W3,
      b3.reshape(1, 1))
    return out[:, 0]


# speedup vs baseline: 49.4463x; 1.0050x over previous
import functools

import jax
import jax.numpy as jnp
from jax import lax
from jax.experimental import pallas as pl
from jax.experimental.pallas import tpu as pltpu
from jax.experimental.pallas import tpu_sc as plsc

_ACT = 14
_CH = 128
_HID = 64
_NC = 2
_NS = 16
_NW = _NC * _NS
_CK = 128
_DEGW = 8


def _sc_mesh():
    return plsc.VectorSubcoreMesh(
        core_axis_name="c", subcore_axis_name="s",
        num_cores=_NC, num_subcores=_NS)


def _make_deg_kernel(N, E):
    ept = E // _NW
    niter = ept // _CK
    rpt = N // _NS

    @functools.partial(
        pl.kernel,
        out_type=jax.ShapeDtypeStruct((N, _NC * _DEGW), jnp.float32),
        mesh=_sc_mesh(),
        scratch_types=[
            pltpu.VMEM((4, _CK), jnp.int32),
            pltpu.VMEM((_CK, _DEGW), jnp.float32),
            pltpu.VMEM_SHARED((N, _DEGW), jnp.float32),
            pltpu.SemaphoreType.DMA,
            pltpu.SemaphoreType.DMA,
        ],
        compiler_params=pltpu.CompilerParams(use_tc_tiling_on_sc=False),
    )
    def deg_kernel(ei_hbm, ones_hbm, zeros_hbm, out_hbm, idx_v, ones_v,
                   deg_sh, isem, asem):
        c = lax.axis_index("c")
        s = lax.axis_index("s")
        wid = c * _NS + s
        pltpu.sync_copy(zeros_hbm.at[pl.ds(s * rpt, rpt), pl.ds(0, _DEGW)],
                        deg_sh.at[pl.ds(s * rpt, rpt)])
        pltpu.sync_copy(ones_hbm, ones_v)
        plsc.subcore_barrier()
        base = wid * ept

        def istart(g):
            pltpu.async_copy(ei_hbm.at[1, pl.ds(base + g * _CK, _CK)],
                             idx_v.at[lax.rem(g, 4)], isem)

        def iwait(g):
            pltpu.make_async_copy(ei_hbm.at[1, pl.ds(0, _CK)],
                                  idx_v.at[lax.rem(g, 4)], isem).wait()

        def astart(g):
            pltpu.async_copy(ones_v, deg_sh.at[idx_v.at[lax.rem(g, 4)]],
                             asem, add=True)

        def await_(g):
            pltpu.make_async_copy(ones_v, deg_sh.at[idx_v.at[lax.rem(g, 4)]],
                                  asem).wait()

        istart(0)
        istart(1)
        istart(2)
        iwait(0)
        iwait(1)

        @pl.loop(0, niter)
        def _(g):
            astart(g)

            @pl.when(g >= 1)
            def _():
                await_(g - 1)

            @pl.when(g + 2 < niter)
            def _():
                iwait(g + 2)

            @pl.when(g + 3 < niter)
            def _():
                istart(g + 3)

        await_(niter - 1)
        plsc.subcore_barrier()
        pltpu.sync_copy(deg_sh.at[pl.ds(s * rpt, rpt)],
                        out_hbm.at[pl.ds(s * rpt, rpt),
                                   pl.ds(c * _DEGW, _DEGW)])

    return deg_kernel


_HCH = _CH // 2


def _make_scatter_kernel(N, E):
    ept = E // _NS
    niter = ept // _CK
    rpt = N // _NS

    @functools.partial(
        pl.kernel,
        out_type=jax.ShapeDtypeStruct((N, _CH), jnp.float32),
        mesh=_sc_mesh(),
        scratch_types=[
            pltpu.VMEM((8, 2, _CK), jnp.int32),
            pltpu.VMEM((8, _CK), jnp.int32),
            pltpu.VMEM((8, _CK, _HCH), jnp.float32),
            pltpu.VMEM_SHARED((N, _HCH), jnp.float32),
            pltpu.SemaphoreType.DMA,
            pltpu.SemaphoreType.DMA,
            pltpu.SemaphoreType.DMA,
        ],
        compiler_params=pltpu.CompilerParams(use_tc_tiling_on_sc=False),
    )
    def scatter_kernel(y2_hbm, ei_hbm, zeros_hbm, out_hbm,
                       ei_v, tr_v, rows_v, s_sh, isem, gsem, asem):
        c = lax.axis_index("c")
        s = lax.axis_index("s")
        pltpu.sync_copy(zeros_hbm.at[pl.ds(s * rpt, rpt)],
                        s_sh.at[pl.ds(s * rpt, rpt)])
        plsc.subcore_barrier()
        base = s * ept
        c2 = jnp.broadcast_to(c.astype(jnp.int32), (16,))

        def istart(g):
            pltpu.async_copy(ei_hbm.at[:, pl.ds(base + g * _CK, _CK)],
                             ei_v.at[lax.rem(g, 8)], isem)

        def iwait(g):
            pltpu.make_async_copy(ei_hbm.at[:, pl.ds(0, _CK)],
                                  ei_v.at[lax.rem(g, 8)], isem).wait()

        def transform(g):
            b = lax.rem(g, 8)
            for k in range(_CK // 16):
                v = ei_v[b, 0, pl.ds(16 * k, 16)]
                tr_v[b, pl.ds(16 * k, 16)] = v * 2 + c2

        def gstart(g):
            b = lax.rem(g, 8)
            pltpu.async_copy(y2_hbm.at[tr_v.at[b]], rows_v.at[b], gsem)

        def gwait(g):
            b = lax.rem(g, 8)
            pltpu.make_async_copy(y2_hbm.at[tr_v.at[b]],
                                  rows_v.at[b], gsem).wait()

        def astart(g):
            b = lax.rem(g, 8)
            pltpu.async_copy(rows_v.at[b], s_sh.at[ei_v.at[b, 1]],
                             asem, add=True)

        def await_(g):
            b = lax.rem(g, 8)
            pltpu.make_async_copy(rows_v.at[b], s_sh.at[ei_v.at[b, 1]],
                                  asem).wait()

        istart(0)
        istart(1)
        istart(2)
        istart(3)
        iwait(0)
        transform(0)
        gstart(0)
        iwait(1)
        transform(1)
        gstart(1)
        iwait(2)
        transform(2)
        gstart(2)

        @pl.loop(0, niter)
        def _(g):
            gwait(g)
            astart(g)

            @pl.when(g >= 2)
            def _():
                await_(g - 2)

            @pl.when(g + 3 < niter)
            def _():
                iwait(g + 3)
                transform(g + 3)
                gstart(g + 3)

            @pl.when(g + 4 < niter)
            def _():
                istart(g + 4)

        @pl.when(niter >= 2)
        def _():
            await_(niter - 2)

        await_(niter - 1)
        plsc.subcore_barrier()
        for ci in range(_NC):
            @pl.when(c == ci)
            def _(ci=ci):
                pltpu.sync_copy(
                    s_sh.at[pl.ds(s * rpt, rpt)],
                    out_hbm.at[pl.ds(s * rpt, rpt), pl.ds(ci * _HCH, _HCH)])

    return scatter_kernel


def _tc1_body(state_ref, wc_ref, degp_ref, y_ref, dis_ref):
    deg = jnp.sum(degp_ref[...], axis=1, keepdims=True) * (1.0 / _DEGW) + 1.0
    dis = lax.rsqrt(deg)
    xw = jnp.dot(state_ref[...], wc_ref[...], preferred_element_type=jnp.float32)
    y_ref[...] = xw * dis
    dis_ref[...] = dis


def _lrelu(v):
    return jnp.where(v > 0, v, 0.01 * v)


def _tc2_body(s_ref, y_ref, st_ref, dis_ref, act_ref, bc_ref,
              w1a_ref, w1b_ref, w1c_ref, b1_ref, w2_ref, b2_ref, w3_ref, b3_ref,
              out_ref, p_s, q_s):
    w1a = w1a_ref[...]
    w1b = w1b_ref[...]
    bc = bc_ref[...]
    for i in range(_ACT):
        pre = s_ref[:, i, :] + y_ref[:, i, :]
        xi = jnp.maximum(
            pre * dis_ref[:, i, :] + bc, 0.0) + st_ref[:, i, :]
        p_s[i, :, :] = jnp.dot(xi, w1a, preferred_element_type=jnp.float32)
        q_s[i, :, :] = jnp.dot(xi, w1b, preferred_element_type=jnp.float32)
    w1c = w1c_ref[...]
    b1 = b1_ref[...]
    w2 = w2_ref[...]
    b2 = b2_ref[...]
    nb = out_ref.shape[0]
    acc = jnp.zeros((nb, _HID), jnp.float32)
    for i in range(_ACT):
        pre = p_s[i, :, :] + q_s[(i + 1) % _ACT, :, :] \
            + act_ref[:, i:i + 1] * w1c + b1
        h = jnp.dot(_lrelu(pre), w2, preferred_element_type=jnp.float32) + b2
        acc = acc + _lrelu(h)
        pre = p_s[i, :, :] + q_s[(i + 2) % _ACT, :, :] \
            + act_ref[:, _ACT + i:_ACT + i + 1] * w1c + b1
        h = jnp.dot(_lrelu(pre), w2, preferred_element_type=jnp.float32) + b2
        acc = acc + _lrelu(h)
    out_ref[...] = jnp.dot(acc, w3_ref[...],
                           preferred_element_type=jnp.float32) + b3_ref[...]


def kernel(state, edge_index, action, Wc, bc, W1, b1, W2, b2, W3, b3):
    N, ch = state.shape
    E = edge_index.shape[1]
    B = action.shape[0]
    zeros = jnp.zeros((N, _HCH), jnp.float32)

    degp = _make_deg_kernel(N, E)(
        edge_index,
        jnp.ones((_CK, _DEGW), jnp.float32),
        zeros,
    )

    rb = 1792
    grid1 = N // rb
    y, dis = pl.pallas_call(
        _tc1_body,
        grid=(grid1,),
        in_specs=[
            pl.BlockSpec((rb, ch), lambda i: (i, 0)),
            pl.BlockSpec((ch, ch), lambda i: (0, 0)),
            pl.BlockSpec((rb, _NC * _DEGW), lambda i: (i, 0)),
        ],
        out_specs=[
            pl.BlockSpec((rb, ch), lambda i: (i, 0)),
            pl.BlockSpec((rb, 1), lambda i: (i, 0)),
        ],
        out_shape=[
            jax.ShapeDtypeStruct((N, ch), jnp.float32),
            jax.ShapeDtypeStruct((N, 1), jnp.float32),
        ],
    )(state, Wc, degp)

    s_full = _make_scatter_kernel(N, E)(
        y.reshape(2 * N, _HCH), edge_index, zeros)

    bb = 128
    grid2 = B // bb
    w1a = W1[:ch]
    w1b = W1[ch:2 * ch]
    w1c = W1[2 * ch:]
    out = pl.pallas_call(
        _tc2_body,
        grid=(grid2,),
        in_specs=[
            pl.BlockSpec((bb, _ACT, ch), lambda i: (i, 0, 0)),
            pl.BlockSpec((bb, _ACT, ch), lambda i: (i, 0, 0)),
            pl.BlockSpec((bb, _ACT, ch), lambda i: (i, 0, 0)),
            pl.BlockSpec((bb, _ACT, 1), lambda i: (i, 0, 0)),
            pl.BlockSpec((bb, 2 * _ACT), lambda i: (i, 0)),
            pl.BlockSpec((1, ch), lambda i: (0, 0)),
            pl.BlockSpec((ch, _HID), lambda i: (0, 0)),
            pl.BlockSpec((ch, _HID), lambda i: (0, 0)),
            pl.BlockSpec((1, _HID), lambda i: (0, 0)),
            pl.BlockSpec((1, _HID), lambda i: (0, 0)),
            pl.BlockSpec((_HID, _HID), lambda i: (0, 0)),
            pl.BlockSpec((1, _HID), lambda i: (0, 0)),
            pl.BlockSpec((_HID, 1), lambda i: (0, 0)),
            pl.BlockSpec((1, 1), lambda i: (0, 0)),
        ],
        out_specs=pl.BlockSpec((bb, 1), lambda i: (i, 0)),
        out_shape=jax.ShapeDtypeStruct((B, 1), jnp.float32),
        scratch_shapes=[
            pltpu.VMEM((_ACT, bb, _HID), jnp.float32),
            pltpu.VMEM((_ACT, bb, _HID), jnp.float32),
        ],
    )(s_full.reshape(B, _ACT, ch),
      y.reshape(B, _ACT, ch),
      state.reshape(B, _ACT, ch),
      dis.reshape(B, _ACT, 1),
      action,
      bc.reshape(1, ch),
      w1a, w1b, w1c,
      b1.reshape(1, _HID),
      W2,
      b2.reshape(1, _HID),
      W3,
      b3.reshape(1, 1))
    return out[:, 0]
